# baseline (speedup 1.0000x reference)
import functools

import jax
import jax.numpy as jnp
from jax import lax
from jax.experimental import pallas as pl
from jax.experimental.pallas import tpu as pltpu
from jax.experimental.pallas import tpu_sc as plsc

N = 10000
E = 320000
DIM = 128
HEADS = 4
DH = DIM // HEADS
RP = 16

NC = 2
NS = 16
L = 16
NW = NC * NS

CH = 96
NCHUNK0 = 128
NCHUNK1 = 88
SLAB = 8
EW0 = NCHUNK0 * CH
EW1 = NCHUNK1 * CH
E_PAD = NS * (EW0 + EW1)
RW = DIM + L
SDW = 8
NPADU = 10000
ROWS_SUB = NPADU // NS

BLKN = 2000
BLKE = E_PAD // 16
BLKC = 2000

_f32 = jnp.float32
_i32 = jnp.int32


def _prep_nodes_body(z_ref, wall_ref, va_ref, zp_ref, sdd_ref):
    zblk = z_ref[...]
    zs = jnp.dot(zblk, wall_ref[...], preferred_element_type=_f32)
    ss = jnp.dot(zblk, va_ref[...], preferred_element_type=_f32)
    zp_ref[...] = jnp.concatenate(
        [zs, ss[:, :HEADS], jnp.zeros((BLKN, RW - DIM - HEADS), _f32)],
        axis=1)
    sdd_ref[...] = jnp.concatenate(
        [ss[:, HEADS:], jnp.zeros((BLKN, SDW - HEADS), _f32)], axis=1)


def _prep_nodes(z, wall, va):
    return pl.pallas_call(
        _prep_nodes_body,
        grid=(N // BLKN,),
        in_specs=[
            pl.BlockSpec((BLKN, DIM), lambda i: (i, 0)),
            pl.BlockSpec((DIM, DIM), lambda i: (0, 0)),
            pl.BlockSpec((DIM, 2 * HEADS), lambda i: (0, 0)),
        ],
        out_specs=[
            pl.BlockSpec((BLKN, RW), lambda i: (i, 0)),
            pl.BlockSpec((BLKN, SDW), lambda i: (i, 0)),
        ],
        out_shape=[
            jax.ShapeDtypeStruct((N, RW), _f32),
            jax.ShapeDtypeStruct((N, SDW), _f32),
        ],
    )(z, wall, va)


def _prep_rel_body(ro_ref, w1t_ref, b1_ref, b2t_ref, c2_ref, out_ref):
    i = pl.program_id(0)
    h1 = jnp.maximum(
        jnp.dot(w1t_ref[...], ro_ref[...], preferred_element_type=_f32)
        + b1_ref[...], 0.0)
    sr = jnp.dot(b2t_ref[...], h1, preferred_element_type=_f32) + c2_ref[...]
    cols = i * BLKE + lax.broadcasted_iota(_i32, (HEADS, BLKE), 1)
    out_ref[...] = jnp.where(cols < E, sr, -1e9)


def _prep_rel(roT, w1t, b1c, b2t, c2c):
    return pl.pallas_call(
        _prep_rel_body,
        grid=(E_PAD // BLKE,),
        in_specs=[
            pl.BlockSpec((2, BLKE), lambda i: (0, i)),
            pl.BlockSpec((RP, 2), lambda i: (0, 0)),
            pl.BlockSpec((RP, 1), lambda i: (0, 0)),
            pl.BlockSpec((HEADS, RP), lambda i: (0, 0)),
            pl.BlockSpec((HEADS, 1), lambda i: (0, 0)),
        ],
        out_specs=pl.BlockSpec((HEADS, BLKE), lambda i: (0, i)),
        out_shape=jax.ShapeDtypeStruct((HEADS, E_PAD), _f32),
    )(roT, w1t, b1c, b2t, c2c)


def _bcast16(v, i):
    idx = jnp.full((L, 1), i, _i32)
    dnums = lax.GatherDimensionNumbers(
        offset_dims=(), collapsed_slice_dims=(0,), start_index_map=(0,))
    return lax.gather(v, idx, dnums, slice_sizes=(1,),
                      mode=lax.GatherScatterMode.PROMISE_IN_BOUNDS)


def _edge_body(zp_hbm, sd_hbm, srelT_hbm, sdd_hbm, u_out,
               sd_big, srel_v, sdd_v, rows_v, u_sh,
               gsem0, gsem1, dsem0, dsem1, rsem0, rsem1, ssem0, ssem1,
               ksem):
    c = lax.axis_index("c")
    s = lax.axis_index("s")
    gsem, dsem = (gsem0, gsem1), (dsem0, dsem1)
    rsem, ssem = (rsem0, rsem1), (ssem0, ssem1)
    ebase = c * (NS * EW0) + s * jnp.where(c == 0, EW0, EW1)
    cbase = ebase // CH
    nchunk = jnp.where(c == 0, NCHUNK0, NCHUNK1)

    def _zrow(r, carry):
        for q in range(RW // L):
            rows_v[0, r, pl.ds(q * L, L)] = jnp.zeros((L,), _f32)
        return carry
    lax.fori_loop(0, CH, _zrow, 0)
    for k in range(ROWS_SUB // CH):
        pltpu.sync_copy(rows_v.at[0],
                        u_sh.at[pl.ds(s * ROWS_SUB + k * CH, CH)])
    rem = ROWS_SUB % CH
    pltpu.sync_copy(
        rows_v.at[0, pl.ds(0, rem)],
        u_sh.at[pl.ds(s * ROWS_SUB + (ROWS_SUB // CH) * CH, rem)])

    lane = lax.iota(_i32, L)

    def _sidx(j, col):
        return sd_big.at[(j // SLAB) % 2, j % SLAB, col]

    def fire_slab(k):
        pltpu.async_copy(sd_hbm.at[pl.ds(cbase + k * SLAB, SLAB)],
                         sd_big.at[k % 2], ksem)

    def wait_slab():
        pltpu.make_async_copy(
            sd_hbm.at[pl.ds(0, SLAB)], sd_big.at[0], ksem).wait()

    def fire_gather(j, b):
        pltpu.async_copy(srelT_hbm.at[:, pl.ds((cbase + j) * CH, CH)],
                         srel_v.at[b], rsem[b])
        pltpu.async_copy(zp_hbm.at[_sidx(j, 0)], rows_v.at[b], gsem[b])
        pltpu.async_copy(sdd_hbm.at[_sidx(j, 1)], sdd_v.at[b], dsem[b])

    def wait_gather(b):
        pltpu.make_async_copy(
            zp_hbm.at[sd_big.at[0, 0, 0]], rows_v.at[b], gsem[b]).wait()
        pltpu.make_async_copy(
            sdd_hbm.at[sd_big.at[0, 0, 1]], sdd_v.at[b], dsem[b]).wait()
        pltpu.make_async_copy(
            srelT_hbm.at[:, pl.ds(0, CH)], srel_v.at[b], rsem[b]).wait()

    def fire_scatter(j, b):
        pltpu.async_copy(rows_v.at[b], u_sh.at[_sidx(j, 1)], ssem[b],
                         add=True)

    def wait_scatter(b):
        pltpu.make_async_copy(
            rows_v.at[b], u_sh.at[sd_big.at[0, 0, 1]], ssem[b]).wait()

    def compute(b):
        for g in range(CH // L):
            eidx = g * L + lane
            wh = []
            for h in range(HEADS):
                ss = plsc.load_gather(
                    rows_v.at[b], [eidx, jnp.full((L,), DIM + h, _i32)])
                sd = plsc.load_gather(
                    sdd_v.at[b], [eidx, jnp.full((L,), h, _i32)])
                x = ss + sd + srel_v[b, h, pl.ds(g * L, L)]
                x = jnp.where(x >= 0.0, x, 0.2 * x)
                w = jnp.exp(x)
                wh.append(w)
                plsc.store_scatter(
                    rows_v.at[b], [eidx, jnp.full((L,), DIM + h, _i32)], w)
            for e16 in range(L):
                e = g * L + e16
                for h in range(HEADS):
                    bc = _bcast16(wh[h], e16)
                    for q in range(DH // L):
                        col = h * DH + q * L
                        rows_v[b, e, pl.ds(col, L)] = (
                            rows_v[b, e, pl.ds(col, L)] * bc)

    plsc.subcore_barrier()
    fire_slab(0)
    wait_slab()
    fire_slab(1)
    fire_gather(0, 0)

    def pair(t, carry):
        for b in (0, 1):
            j = 2 * t + b
            wait_gather(b)

            @pl.when(j + 1 < nchunk)
            def _prefetch():
                @pl.when(j >= 1)
                def _drain():
                    wait_scatter(1 - b)

                @pl.when((j + 1) % SLAB == 0)
                def _settle():
                    wait_slab()
                fire_gather(j + 1, 1 - b)

            @pl.when((j % SLAB == 3) & (j >= SLAB) & (j + 5 < nchunk))
            def _slab():
                fire_slab(j // SLAB + 1)

            compute(b)
            fire_scatter(j, b)
        return carry

    lax.fori_loop(0, nchunk // 2, pair, 0)
    wait_scatter(0)
    wait_scatter(1)
    plsc.subcore_barrier()
    pltpu.sync_copy(u_sh.at[pl.ds(s * ROWS_SUB, ROWS_SUB)],
                    u_out.at[c, pl.ds(s * ROWS_SUB, ROWS_SUB)])


def _edge_phase(zp, sd, srelT, sdd):
    mesh = plsc.VectorSubcoreMesh(
        core_axis_name="c", subcore_axis_name="s",
        num_cores=NC, num_subcores=NS)
    return pl.kernel(
        _edge_body,
        out_type=jax.ShapeDtypeStruct((NC, NPADU, RW), _f32),
        mesh=mesh,
        compiler_params=pltpu.CompilerParams(
            needs_layout_passes=False, use_tc_tiling_on_sc=False),
        scratch_types=[
            pltpu.VMEM((2, SLAB, 2, CH), _i32),
            pltpu.VMEM((2, HEADS, CH), _f32),
            pltpu.VMEM((2, CH, SDW), _f32),
            pltpu.VMEM((2, CH, RW), _f32),
            pltpu.VMEM_SHARED((NPADU, RW), _f32),
            pltpu.SemaphoreType.DMA,
            pltpu.SemaphoreType.DMA,
            pltpu.SemaphoreType.DMA,
            pltpu.SemaphoreType.DMA,
            pltpu.SemaphoreType.DMA,
            pltpu.SemaphoreType.DMA,
            pltpu.SemaphoreType.DMA,
            pltpu.SemaphoreType.DMA,
            pltpu.SemaphoreType.DMA,
        ],
    )(zp, sd, srelT, sdd)


def _final_body(u_ref, z_ref, pw_ref, pb_ref, ind_ref, out_ref):
    u = u_ref[0] + u_ref[1]
    d = u[:, DIM:DIM + HEADS]
    dinv = 1.0 / (d + 1e-6)
    dexp = jnp.dot(dinv, ind_ref[...], preferred_element_type=_f32)
    agg = u[:, :DIM] * dexp
    out_ref[...] = (
        jnp.dot(agg, pw_ref[...], preferred_element_type=_f32)
        + pb_ref[...] + z_ref[...])


def _finalize(u2, z, pw, pbr, ind):
    return pl.pallas_call(
        _final_body,
        grid=(N // BLKC,),
        in_specs=[
            pl.BlockSpec((NC, BLKC, RW), lambda i: (0, i, 0)),
            pl.BlockSpec((BLKC, DIM), lambda i: (i, 0)),
            pl.BlockSpec((DIM, DIM), lambda i: (0, 0)),
            pl.BlockSpec((1, DIM), lambda i: (0, 0)),
            pl.BlockSpec((HEADS, DIM), lambda i: (0, 0)),
        ],
        out_specs=pl.BlockSpec((BLKC, DIM), lambda i: (i, 0)),
        out_shape=jax.ShapeDtypeStruct((N, DIM), _f32),
    )(u2, z, pw, pbr, ind)


def kernel(z, edges, rel_offsets, W, a, rel_W1, rel_b1, rel_W2, rel_b2,
           proj_W, proj_b):
    wall = jnp.transpose(W, (1, 0, 2)).reshape(DIM, DIM)
    a_src, a_dst, a_rel = a[:, :DH], a[:, DH:2 * DH], a[:, 2 * DH:]
    va = jnp.concatenate([jnp.einsum('hdk,hk->dh', W, a_src),
                          jnp.einsum('hdk,hk->dh', W, a_dst)], axis=1)
    b2 = rel_W2 @ a_rel.T
    c2 = rel_b2 @ a_rel.T
    ind = jnp.repeat(jnp.eye(HEADS, dtype=_f32), DH, axis=1)

    roT = jnp.pad(rel_offsets, ((0, E_PAD - E), (0, 0))).T
    pad_i = jnp.zeros((E_PAD - E,), _i32)
    src = jnp.concatenate([edges[:, 0], pad_i])
    dst = jnp.concatenate([edges[:, 1], pad_i])
    sd = jnp.stack([src.reshape(-1, CH), dst.reshape(-1, CH)], axis=1)

    zp, sdd = _prep_nodes(z, wall, va)
    srelT = _prep_rel(roT, rel_W1.T, rel_b1.reshape(RP, 1),
                      b2.T, c2.reshape(HEADS, 1))
    u2 = _edge_phase(zp, sd, srelT, sdd)
    return _finalize(u2, z, proj_W, proj_b.reshape(1, DIM), ind)


# speedup vs baseline: 1.3591x; 1.3591x over previous
import functools

import jax
import jax.numpy as jnp
from jax import lax
from jax.experimental import pallas as pl
from jax.experimental.pallas import tpu as pltpu
from jax.experimental.pallas import tpu_sc as plsc

N = 10000
E = 320000
DIM = 128
HEADS = 4
DH = DIM // HEADS
RP = 16

NC = 2
NS = 16
L = 16
NW = NC * NS

CH = 96
NCHUNK0 = 128
NCHUNK1 = 84
EW0 = NCHUNK0 * CH
EW1 = NCHUNK1 * CH
E_PAD = NS * (EW0 + EW1)
RW = DIM + L
SDW = 8
NPADU = 10000
ROWS_SUB = NPADU // NS

BLKN = 2000
BLKE = E_PAD // 16
BLKC = 2000

_f32 = jnp.float32
_i32 = jnp.int32


def _prep_nodes_body(z_ref, wall_ref, va_ref, zp_ref, sdd_ref):
    zblk = z_ref[...]
    zs = jnp.dot(zblk, wall_ref[...], preferred_element_type=_f32)
    ss = jnp.dot(zblk, va_ref[...], preferred_element_type=_f32)
    zp_ref[...] = jnp.concatenate(
        [zs, ss[:, :HEADS], jnp.zeros((BLKN, RW - DIM - HEADS), _f32)],
        axis=1)
    sdd_ref[...] = jnp.concatenate(
        [ss[:, HEADS:], jnp.zeros((BLKN, SDW - HEADS), _f32)], axis=1)


def _prep_nodes(z, wall, va):
    return pl.pallas_call(
        _prep_nodes_body,
        grid=(N // BLKN,),
        in_specs=[
            pl.BlockSpec((BLKN, DIM), lambda i: (i, 0)),
            pl.BlockSpec((DIM, DIM), lambda i: (0, 0)),
            pl.BlockSpec((DIM, 2 * HEADS), lambda i: (0, 0)),
        ],
        out_specs=[
            pl.BlockSpec((BLKN, RW), lambda i: (i, 0)),
            pl.BlockSpec((BLKN, SDW), lambda i: (i, 0)),
        ],
        out_shape=[
            jax.ShapeDtypeStruct((N, RW), _f32),
            jax.ShapeDtypeStruct((N, SDW), _f32),
        ],
    )(z, wall, va)


def _prep_rel_body(ro_ref, w1t_ref, b1_ref, b2t_ref, c2_ref, out_ref):
    i = pl.program_id(0)
    h1 = jnp.maximum(
        jnp.dot(w1t_ref[...], ro_ref[...], preferred_element_type=_f32)
        + b1_ref[...], 0.0)
    sr = jnp.dot(b2t_ref[...], h1, preferred_element_type=_f32) + c2_ref[...]
    cols = i * BLKE + lax.broadcasted_iota(_i32, (HEADS, BLKE), 1)
    out_ref[...] = jnp.where(cols < E, sr, -1e9)


def _prep_rel(roT, w1t, b1c, b2t, c2c):
    return pl.pallas_call(
        _prep_rel_body,
        grid=(E_PAD // BLKE,),
        in_specs=[
            pl.BlockSpec((2, BLKE), lambda i: (0, i)),
            pl.BlockSpec((RP, 2), lambda i: (0, 0)),
            pl.BlockSpec((RP, 1), lambda i: (0, 0)),
            pl.BlockSpec((HEADS, RP), lambda i: (0, 0)),
            pl.BlockSpec((HEADS, 1), lambda i: (0, 0)),
        ],
        out_specs=pl.BlockSpec((HEADS, BLKE), lambda i: (0, i)),
        out_shape=jax.ShapeDtypeStruct((HEADS, E_PAD), _f32),
    )(roT, w1t, b1c, b2t, c2c)


def _bcast16(v, i):
    idx = jnp.full((L, 1), i, _i32)
    dnums = lax.GatherDimensionNumbers(
        offset_dims=(), collapsed_slice_dims=(0,), start_index_map=(0,))
    return lax.gather(v, idx, dnums, slice_sizes=(1,),
                      mode=lax.GatherScatterMode.PROMISE_IN_BOUNDS)


def _edge_body(zp_hbm, sd_hbm, srelT_hbm, sdd_hbm, u_out,
               sd_v, srel_v, sdd_v, rows_v, u_sh,
               gsem0, gsem1, dsem0, dsem1, rsem0, rsem1, ssem0, ssem1):
    c = lax.axis_index("c")
    s = lax.axis_index("s")
    gsem, dsem = (gsem0, gsem1), (dsem0, dsem1)
    rsem, ssem = (rsem0, rsem1), (ssem0, ssem1)
    ebase = c * (NS * EW0) + s * jnp.where(c == 0, EW0, EW1)
    cbase = ebase // CH
    nchunk = jnp.where(c == 0, NCHUNK0, NCHUNK1)

    def _zrow(r, carry):
        for q in range(RW // L):
            rows_v[0, r, pl.ds(q * L, L)] = jnp.zeros((L,), _f32)
        return carry
    lax.fori_loop(0, CH, _zrow, 0)
    for k in range(ROWS_SUB // CH):
        pltpu.sync_copy(rows_v.at[0],
                        u_sh.at[pl.ds(s * ROWS_SUB + k * CH, CH)])
    rem = ROWS_SUB % CH
    pltpu.sync_copy(
        rows_v.at[0, pl.ds(0, rem)],
        u_sh.at[pl.ds(s * ROWS_SUB + (ROWS_SUB // CH) * CH, rem)])

    lane = lax.iota(_i32, L)

    def stage(j, b):
        pltpu.sync_copy(sd_hbm.at[cbase + j], sd_v.at[b])
        pltpu.async_copy(srelT_hbm.at[:, pl.ds((cbase + j) * CH, CH)],
                         srel_v.at[b], rsem[b])

    def fire_gather(b):
        pltpu.async_copy(zp_hbm.at[sd_v.at[b, 0]], rows_v.at[b], gsem[b])
        pltpu.async_copy(sdd_hbm.at[sd_v.at[b, 1]], sdd_v.at[b], dsem[b])

    def wait_gather(b):
        pltpu.make_async_copy(
            zp_hbm.at[sd_v.at[b, 0]], rows_v.at[b], gsem[b]).wait()
        pltpu.make_async_copy(
            sdd_hbm.at[sd_v.at[b, 1]], sdd_v.at[b], dsem[b]).wait()
        pltpu.make_async_copy(
            srelT_hbm.at[:, pl.ds(0, CH)], srel_v.at[b], rsem[b]).wait()

    def fire_scatter(b):
        pltpu.async_copy(rows_v.at[b], u_sh.at[sd_v.at[b, 1]], ssem[b],
                         add=True)

    def wait_scatter(b):
        pltpu.make_async_copy(
            rows_v.at[b], u_sh.at[sd_v.at[b, 1]], ssem[b]).wait()

    def compute(b):
        for g in range(CH // L):
            eidx = g * L + lane
            wh = []
            for h in range(HEADS):
                ss = plsc.load_gather(
                    rows_v.at[b], [eidx, jnp.full((L,), DIM + h, _i32)])
                sd = plsc.load_gather(
                    sdd_v.at[b], [eidx, jnp.full((L,), h, _i32)])
                x = ss + sd + srel_v[b, h, pl.ds(g * L, L)]
                x = jnp.where(x >= 0.0, x, 0.2 * x)
                w = jnp.exp(x)
                wh.append(w)
                plsc.store_scatter(
                    rows_v.at[b], [eidx, jnp.full((L,), DIM + h, _i32)], w)
            for e16 in range(L):
                e = g * L + e16
                for h in range(HEADS):
                    bc = _bcast16(wh[h], e16)
                    for q in range(DH // L):
                        col = h * DH + q * L
                        rows_v[b, e, pl.ds(col, L)] = (
                            rows_v[b, e, pl.ds(col, L)] * bc)

    plsc.subcore_barrier()
    stage(0, 0)
    fire_gather(0)

    def pair(t, carry):
        for b in (0, 1):
            j = 2 * t + b
            wait_gather(b)

            @pl.when(j + 1 < nchunk)
            def _prefetch():
                @pl.when(j >= 1)
                def _drain():
                    wait_scatter(1 - b)
                stage(j + 1, 1 - b)
                fire_gather(1 - b)

            compute(b)
            fire_scatter(b)
        return carry

    lax.fori_loop(0, nchunk // 2, pair, 0)
    wait_scatter(0)
    wait_scatter(1)
    plsc.subcore_barrier()
    pltpu.sync_copy(u_sh.at[pl.ds(s * ROWS_SUB, ROWS_SUB)],
                    u_out.at[c, pl.ds(s * ROWS_SUB, ROWS_SUB)])


def _edge_phase(zp, sd, srelT, sdd):
    mesh = plsc.VectorSubcoreMesh(
        core_axis_name="c", subcore_axis_name="s",
        num_cores=NC, num_subcores=NS)
    return pl.kernel(
        _edge_body,
        out_type=jax.ShapeDtypeStruct((NC, NPADU, RW), _f32),
        mesh=mesh,
        compiler_params=pltpu.CompilerParams(
            needs_layout_passes=False, use_tc_tiling_on_sc=False),
        scratch_types=[
            pltpu.VMEM((2, 2, CH), _i32),
            pltpu.VMEM((2, HEADS, CH), _f32),
            pltpu.VMEM((2, CH, SDW), _f32),
            pltpu.VMEM((2, CH, RW), _f32),
            pltpu.VMEM_SHARED((NPADU, RW), _f32),
            pltpu.SemaphoreType.DMA,
            pltpu.SemaphoreType.DMA,
            pltpu.SemaphoreType.DMA,
            pltpu.SemaphoreType.DMA,
            pltpu.SemaphoreType.DMA,
            pltpu.SemaphoreType.DMA,
            pltpu.SemaphoreType.DMA,
            pltpu.SemaphoreType.DMA,
        ],
    )(zp, sd, srelT, sdd)


def _final_body(u_ref, z_ref, pw_ref, pb_ref, ind_ref, out_ref):
    u = u_ref[0] + u_ref[1]
    d = u[:, DIM:DIM + HEADS]
    dinv = 1.0 / (d + 1e-6)
    dexp = jnp.dot(dinv, ind_ref[...], preferred_element_type=_f32)
    agg = u[:, :DIM] * dexp
    out_ref[...] = (
        jnp.dot(agg, pw_ref[...], preferred_element_type=_f32)
        + pb_ref[...] + z_ref[...])


def _finalize(u2, z, pw, pbr, ind):
    return pl.pallas_call(
        _final_body,
        grid=(N // BLKC,),
        in_specs=[
            pl.BlockSpec((NC, BLKC, RW), lambda i: (0, i, 0)),
            pl.BlockSpec((BLKC, DIM), lambda i: (i, 0)),
            pl.BlockSpec((DIM, DIM), lambda i: (0, 0)),
            pl.BlockSpec((1, DIM), lambda i: (0, 0)),
            pl.BlockSpec((HEADS, DIM), lambda i: (0, 0)),
        ],
        out_specs=pl.BlockSpec((BLKC, DIM), lambda i: (i, 0)),
        out_shape=jax.ShapeDtypeStruct((N, DIM), _f32),
    )(u2, z, pw, pbr, ind)


def kernel(z, edges, rel_offsets, W, a, rel_W1, rel_b1, rel_W2, rel_b2,
           proj_W, proj_b):
    wall = jnp.transpose(W, (1, 0, 2)).reshape(DIM, DIM)
    a_src, a_dst, a_rel = a[:, :DH], a[:, DH:2 * DH], a[:, 2 * DH:]
    va = jnp.concatenate([jnp.einsum('hdk,hk->dh', W, a_src),
                          jnp.einsum('hdk,hk->dh', W, a_dst)], axis=1)
    b2 = rel_W2 @ a_rel.T
    c2 = rel_b2 @ a_rel.T
    ind = jnp.repeat(jnp.eye(HEADS, dtype=_f32), DH, axis=1)

    roT = jnp.pad(rel_offsets, ((0, E_PAD - E), (0, 0))).T
    pad_i = jnp.zeros((E_PAD - E,), _i32)
    src = jnp.concatenate([edges[:, 0], pad_i])
    dst = jnp.concatenate([edges[:, 1], pad_i])
    sd = jnp.stack([src.reshape(-1, CH), dst.reshape(-1, CH)], axis=1)

    zp, sdd = _prep_nodes(z, wall, va)
    srelT = _prep_rel(roT, rel_W1.T, rel_b1.reshape(RP, 1),
                      b2.T, c2.reshape(HEADS, 1))
    u2 = _edge_phase(zp, sd, srelT, sdd)
    return _finalize(u2, z, proj_W, proj_b.reshape(1, DIM), ind)


# speedup vs baseline: 1.4531x; 1.0692x over previous
import functools

import jax
import jax.numpy as jnp
from jax import lax
from jax.experimental import pallas as pl
from jax.experimental.pallas import tpu as pltpu
from jax.experimental.pallas import tpu_sc as plsc

N = 10000
E = 320000
DIM = 128
HEADS = 4
DH = DIM // HEADS
RP = 16

NC = 2
NS = 16
L = 16
NW = NC * NS

CH = 112
NCHUNK0 = 108
NCHUNK1 = 72
EW0 = NCHUNK0 * CH
EW1 = NCHUNK1 * CH
E_PAD = NS * (EW0 + EW1)
RW = DIM + L
SDW = 8
NPADU = 10000
ROWS_SUB = NPADU // NS

BLKN = 2000
BLKE = E_PAD // 8
BLKC = 2000

_f32 = jnp.float32
_i32 = jnp.int32


def _prep_nodes_body(z_ref, wall_ref, va_ref, zp_ref, sdd_ref):
    zblk = z_ref[...]
    zs = jnp.dot(zblk, wall_ref[...], preferred_element_type=_f32)
    ss = jnp.dot(zblk, va_ref[...], preferred_element_type=_f32)
    zp_ref[...] = jnp.concatenate(
        [zs, ss[:, :HEADS], jnp.zeros((BLKN, RW - DIM - HEADS), _f32)],
        axis=1)
    sdd_ref[...] = jnp.concatenate(
        [ss[:, HEADS:], jnp.zeros((BLKN, SDW - HEADS), _f32)], axis=1)


def _prep_nodes(z, wall, va):
    return pl.pallas_call(
        _prep_nodes_body,
        grid=(N // BLKN,),
        in_specs=[
            pl.BlockSpec((BLKN, DIM), lambda i: (i, 0)),
            pl.BlockSpec((DIM, DIM), lambda i: (0, 0)),
            pl.BlockSpec((DIM, 2 * HEADS), lambda i: (0, 0)),
        ],
        out_specs=[
            pl.BlockSpec((BLKN, RW), lambda i: (i, 0)),
            pl.BlockSpec((BLKN, SDW), lambda i: (i, 0)),
        ],
        out_shape=[
            jax.ShapeDtypeStruct((N, RW), _f32),
            jax.ShapeDtypeStruct((N, SDW), _f32),
        ],
    )(z, wall, va)


def _prep_rel_body(ro_ref, w1t_ref, b1_ref, b2t_ref, c2_ref, out_ref):
    i = pl.program_id(0)
    h1 = jnp.maximum(
        jnp.dot(w1t_ref[...], ro_ref[...], preferred_element_type=_f32)
        + b1_ref[...], 0.0)
    sr = jnp.dot(b2t_ref[...], h1, preferred_element_type=_f32) + c2_ref[...]
    cols = i * BLKE + lax.broadcasted_iota(_i32, (HEADS, BLKE), 1)
    out_ref[...] = jnp.where(cols < E, sr, -1e9)


def _prep_rel(roT, w1t, b1c, b2t, c2c):
    return pl.pallas_call(
        _prep_rel_body,
        grid=(E_PAD // BLKE,),
        in_specs=[
            pl.BlockSpec((2, BLKE), lambda i: (0, i)),
            pl.BlockSpec((RP, 2), lambda i: (0, 0)),
            pl.BlockSpec((RP, 1), lambda i: (0, 0)),
            pl.BlockSpec((HEADS, RP), lambda i: (0, 0)),
            pl.BlockSpec((HEADS, 1), lambda i: (0, 0)),
        ],
        out_specs=pl.BlockSpec((HEADS, BLKE), lambda i: (0, i)),
        out_shape=jax.ShapeDtypeStruct((HEADS, E_PAD), _f32),
    )(roT, w1t, b1c, b2t, c2c)


def _bcast16(v, i):
    idx = jnp.full((L, 1), i, _i32)
    dnums = lax.GatherDimensionNumbers(
        offset_dims=(), collapsed_slice_dims=(0,), start_index_map=(0,))
    return lax.gather(v, idx, dnums, slice_sizes=(1,),
                      mode=lax.GatherScatterMode.PROMISE_IN_BOUNDS)


def _edge_body(zp_hbm, sd_hbm, srelT_hbm, sdd_hbm, u_out,
               sd_v, srel_v, sdd_v, rows_v, u_sh,
               gsem0, gsem1, dsem0, dsem1, rsem0, rsem1, ssem0, ssem1):
    c = lax.axis_index("c")
    s = lax.axis_index("s")
    gsem, dsem = (gsem0, gsem1), (dsem0, dsem1)
    rsem, ssem = (rsem0, rsem1), (ssem0, ssem1)
    ebase = c * (NS * EW0) + s * jnp.where(c == 0, EW0, EW1)
    cbase = ebase // CH
    nchunk = jnp.where(c == 0, NCHUNK0, NCHUNK1)

    def _zrow(r, carry):
        for q in range(RW // L):
            rows_v[0, r, pl.ds(q * L, L)] = jnp.zeros((L,), _f32)
        return carry
    lax.fori_loop(0, CH, _zrow, 0)
    for k in range(ROWS_SUB // CH):
        pltpu.sync_copy(rows_v.at[0],
                        u_sh.at[pl.ds(s * ROWS_SUB + k * CH, CH)])
    rem = ROWS_SUB % CH
    pltpu.sync_copy(
        rows_v.at[0, pl.ds(0, rem)],
        u_sh.at[pl.ds(s * ROWS_SUB + (ROWS_SUB // CH) * CH, rem)])

    lane = lax.iota(_i32, L)

    def stage(j, b):
        pltpu.sync_copy(sd_hbm.at[cbase + j], sd_v.at[b])
        pltpu.async_copy(srelT_hbm.at[:, pl.ds((cbase + j) * CH, CH)],
                         srel_v.at[b], rsem[b])

    def fire_gather(b):
        pltpu.async_copy(zp_hbm.at[sd_v.at[b, 0]], rows_v.at[b], gsem[b])
        pltpu.async_copy(sdd_hbm.at[sd_v.at[b, 1]], sdd_v.at[b], dsem[b])

    def wait_gather(b):
        pltpu.make_async_copy(
            zp_hbm.at[sd_v.at[b, 0]], rows_v.at[b], gsem[b]).wait()
        pltpu.make_async_copy(
            sdd_hbm.at[sd_v.at[b, 1]], sdd_v.at[b], dsem[b]).wait()
        pltpu.make_async_copy(
            srelT_hbm.at[:, pl.ds(0, CH)], srel_v.at[b], rsem[b]).wait()

    def fire_scatter(b):
        pltpu.async_copy(rows_v.at[b], u_sh.at[sd_v.at[b, 1]], ssem[b],
                         add=True)

    def wait_scatter(b):
        pltpu.make_async_copy(
            rows_v.at[b], u_sh.at[sd_v.at[b, 1]], ssem[b]).wait()

    def compute(b):
        for g in range(CH // L):
            eidx = g * L + lane
            wh = []
            for h in range(HEADS):
                ss = plsc.load_gather(
                    rows_v.at[b], [eidx, jnp.full((L,), DIM + h, _i32)])
                sd = plsc.load_gather(
                    sdd_v.at[b], [eidx, jnp.full((L,), h, _i32)])
                x = ss + sd + srel_v[b, h, pl.ds(g * L, L)]
                x = jnp.where(x >= 0.0, x, 0.2 * x)
                w = jnp.exp(x)
                wh.append(w)
                plsc.store_scatter(
                    rows_v.at[b], [eidx, jnp.full((L,), DIM + h, _i32)], w)
            for e16 in range(L):
                e = g * L + e16
                for h in range(HEADS):
                    bc = _bcast16(wh[h], e16)
                    for q in range(DH // L):
                        col = h * DH + q * L
                        rows_v[b, e, pl.ds(col, L)] = (
                            rows_v[b, e, pl.ds(col, L)] * bc)

    plsc.subcore_barrier()
    stage(0, 0)
    fire_gather(0)

    def pair(t, carry):
        for b in (0, 1):
            j = 2 * t + b
            wait_gather(b)

            @pl.when(j + 1 < nchunk)
            def _prefetch():
                @pl.when(j >= 1)
                def _drain():
                    wait_scatter(1 - b)
                stage(j + 1, 1 - b)
                fire_gather(1 - b)

            compute(b)
            fire_scatter(b)
        return carry

    lax.fori_loop(0, nchunk // 2, pair, 0)
    wait_scatter(0)
    wait_scatter(1)
    plsc.subcore_barrier()
    pltpu.sync_copy(u_sh.at[pl.ds(s * ROWS_SUB, ROWS_SUB)],
                    u_out.at[c, pl.ds(s * ROWS_SUB, ROWS_SUB)])


def _edge_phase(zp, sd, srelT, sdd):
    mesh = plsc.VectorSubcoreMesh(
        core_axis_name="c", subcore_axis_name="s",
        num_cores=NC, num_subcores=NS)
    return pl.kernel(
        _edge_body,
        out_type=jax.ShapeDtypeStruct((NC, NPADU, RW), _f32),
        mesh=mesh,
        compiler_params=pltpu.CompilerParams(
            needs_layout_passes=False, use_tc_tiling_on_sc=False),
        scratch_types=[
            pltpu.VMEM((2, 2, CH), _i32),
            pltpu.VMEM((2, HEADS, CH), _f32),
            pltpu.VMEM((2, CH, SDW), _f32),
            pltpu.VMEM((2, CH, RW), _f32),
            pltpu.VMEM_SHARED((NPADU, RW), _f32),
            pltpu.SemaphoreType.DMA,
            pltpu.SemaphoreType.DMA,
            pltpu.SemaphoreType.DMA,
            pltpu.SemaphoreType.DMA,
            pltpu.SemaphoreType.DMA,
            pltpu.SemaphoreType.DMA,
            pltpu.SemaphoreType.DMA,
            pltpu.SemaphoreType.DMA,
        ],
    )(zp, sd, srelT, sdd)


def _final_body(u_ref, z_ref, pw_ref, pb_ref, ind_ref, out_ref):
    u = u_ref[0] + u_ref[1]
    d = u[:, DIM:DIM + HEADS]
    dinv = 1.0 / (d + 1e-6)
    dexp = jnp.dot(dinv, ind_ref[...], preferred_element_type=_f32)
    agg = u[:, :DIM] * dexp
    out_ref[...] = (
        jnp.dot(agg, pw_ref[...], preferred_element_type=_f32)
        + pb_ref[...] + z_ref[...])


def _finalize(u2, z, pw, pbr, ind):
    return pl.pallas_call(
        _final_body,
        grid=(N // BLKC,),
        in_specs=[
            pl.BlockSpec((NC, BLKC, RW), lambda i: (0, i, 0)),
            pl.BlockSpec((BLKC, DIM), lambda i: (i, 0)),
            pl.BlockSpec((DIM, DIM), lambda i: (0, 0)),
            pl.BlockSpec((1, DIM), lambda i: (0, 0)),
            pl.BlockSpec((HEADS, DIM), lambda i: (0, 0)),
        ],
        out_specs=pl.BlockSpec((BLKC, DIM), lambda i: (i, 0)),
        out_shape=jax.ShapeDtypeStruct((N, DIM), _f32),
    )(u2, z, pw, pbr, ind)


def kernel(z, edges, rel_offsets, W, a, rel_W1, rel_b1, rel_W2, rel_b2,
           proj_W, proj_b):
    wall = jnp.transpose(W, (1, 0, 2)).reshape(DIM, DIM)
    a_src, a_dst, a_rel = a[:, :DH], a[:, DH:2 * DH], a[:, 2 * DH:]
    va = jnp.concatenate([jnp.einsum('hdk,hk->dh', W, a_src),
                          jnp.einsum('hdk,hk->dh', W, a_dst)], axis=1)
    b2 = rel_W2 @ a_rel.T
    c2 = rel_b2 @ a_rel.T
    ind = jnp.repeat(jnp.eye(HEADS, dtype=_f32), DH, axis=1)

    roT = jnp.pad(rel_offsets, ((0, E_PAD - E), (0, 0))).T
    pad_i = jnp.zeros((E_PAD - E,), _i32)
    src = jnp.concatenate([edges[:, 0], pad_i])
    dst = jnp.concatenate([edges[:, 1], pad_i])
    sd = jnp.stack([src.reshape(-1, CH), dst.reshape(-1, CH)], axis=1)

    zp, sdd = _prep_nodes(z, wall, va)
    srelT = _prep_rel(roT, rel_W1.T, rel_b1.reshape(RP, 1),
                      b2.T, c2.reshape(HEADS, 1))
    u2 = _edge_phase(zp, sd, srelT, sdd)
    return _finalize(u2, z, proj_W, proj_b.reshape(1, DIM), ind)


# speedup vs baseline: 1.5608x; 1.0741x over previous
import functools

import jax
import jax.numpy as jnp
from jax import lax
from jax.experimental import pallas as pl
from jax.experimental.pallas import tpu as pltpu
from jax.experimental.pallas import tpu_sc as plsc

N = 10000
E = 320000
DIM = 128
HEADS = 4
DH = DIM // HEADS
RP = 16

NC = 2
NS = 16
L = 16
NW = NC * NS

CH = 112
NCHUNK0 = 98
NCHUNK1 = 82
EW0 = NCHUNK0 * CH
EW1 = NCHUNK1 * CH
E_PAD = NS * (EW0 + EW1)
RW = DIM + L
SDW = 8
NPADU = 10000
ROWS_SUB = NPADU // NS

BLKN = 2000
BLKE = E_PAD // 8
BLKC = 2000

_f32 = jnp.float32
_i32 = jnp.int32


def _prep_nodes_body(z_ref, wall_ref, va_ref, zp_ref, sdd_ref):
    zblk = z_ref[...]
    zs = jnp.dot(zblk, wall_ref[...], preferred_element_type=_f32)
    ss = jnp.dot(zblk, va_ref[...], preferred_element_type=_f32)
    zp_ref[...] = jnp.concatenate(
        [zs, ss[:, :HEADS], jnp.zeros((BLKN, RW - DIM - HEADS), _f32)],
        axis=1)
    sdd_ref[...] = jnp.concatenate(
        [ss[:, HEADS:], jnp.zeros((BLKN, SDW - HEADS), _f32)], axis=1)


def _prep_nodes(z, wall, va):
    return pl.pallas_call(
        _prep_nodes_body,
        grid=(N // BLKN,),
        in_specs=[
            pl.BlockSpec((BLKN, DIM), lambda i: (i, 0)),
            pl.BlockSpec((DIM, DIM), lambda i: (0, 0)),
            pl.BlockSpec((DIM, 2 * HEADS), lambda i: (0, 0)),
        ],
        out_specs=[
            pl.BlockSpec((BLKN, RW), lambda i: (i, 0)),
            pl.BlockSpec((BLKN, SDW), lambda i: (i, 0)),
        ],
        out_shape=[
            jax.ShapeDtypeStruct((N, RW), _f32),
            jax.ShapeDtypeStruct((N, SDW), _f32),
        ],
    )(z, wall, va)


def _prep_rel_body(ro_ref, w1t_ref, b1_ref, b2t_ref, c2_ref, out_ref):
    i = pl.program_id(0)
    h1 = jnp.maximum(
        jnp.dot(w1t_ref[...], ro_ref[...], preferred_element_type=_f32)
        + b1_ref[...], 0.0)
    sr = jnp.dot(b2t_ref[...], h1, preferred_element_type=_f32) + c2_ref[...]
    cols = i * BLKE + lax.broadcasted_iota(_i32, (HEADS, BLKE), 1)
    out_ref[...] = jnp.where(cols < E, sr, -1e9)


def _prep_rel(roT, w1t, b1c, b2t, c2c):
    return pl.pallas_call(
        _prep_rel_body,
        grid=(E_PAD // BLKE,),
        in_specs=[
            pl.BlockSpec((2, BLKE), lambda i: (0, i)),
            pl.BlockSpec((RP, 2), lambda i: (0, 0)),
            pl.BlockSpec((RP, 1), lambda i: (0, 0)),
            pl.BlockSpec((HEADS, RP), lambda i: (0, 0)),
            pl.BlockSpec((HEADS, 1), lambda i: (0, 0)),
        ],
        out_specs=pl.BlockSpec((HEADS, BLKE), lambda i: (0, i)),
        out_shape=jax.ShapeDtypeStruct((HEADS, E_PAD), _f32),
    )(roT, w1t, b1c, b2t, c2c)


def _bcast16(v, i):
    idx = jnp.full((L, 1), i, _i32)
    dnums = lax.GatherDimensionNumbers(
        offset_dims=(), collapsed_slice_dims=(0,), start_index_map=(0,))
    return lax.gather(v, idx, dnums, slice_sizes=(1,),
                      mode=lax.GatherScatterMode.PROMISE_IN_BOUNDS)


def _edge_body(zp_hbm, sd_hbm, srelT_hbm, sdd_hbm, u_out,
               sd_v, srel_v, sdd_v, rows_v, u_sh,
               gsem0, gsem1, dsem0, dsem1, rsem0, rsem1, ssem0, ssem1):
    c = lax.axis_index("c")
    s = lax.axis_index("s")
    gsem, dsem = (gsem0, gsem1), (dsem0, dsem1)
    rsem, ssem = (rsem0, rsem1), (ssem0, ssem1)
    ebase = c * (NS * EW0) + s * jnp.where(c == 0, EW0, EW1)
    cbase = ebase // CH
    nchunk = jnp.where(c == 0, NCHUNK0, NCHUNK1)

    def _zrow(r, carry):
        for q in range(RW // L):
            rows_v[0, r, pl.ds(q * L, L)] = jnp.zeros((L,), _f32)
        return carry
    lax.fori_loop(0, CH, _zrow, 0)
    for k in range(ROWS_SUB // CH):
        pltpu.sync_copy(rows_v.at[0],
                        u_sh.at[pl.ds(s * ROWS_SUB + k * CH, CH)])
    rem = ROWS_SUB % CH
    pltpu.sync_copy(
        rows_v.at[0, pl.ds(0, rem)],
        u_sh.at[pl.ds(s * ROWS_SUB + (ROWS_SUB // CH) * CH, rem)])

    lane = lax.iota(_i32, L)

    def stage(j, b):
        pltpu.sync_copy(sd_hbm.at[cbase + j], sd_v.at[b])
        pltpu.async_copy(srelT_hbm.at[:, pl.ds((cbase + j) * CH, CH)],
                         srel_v.at[b], rsem[b])

    def fire_gather(b):
        pltpu.async_copy(zp_hbm.at[sd_v.at[b, 0]], rows_v.at[b], gsem[b])
        pltpu.async_copy(sdd_hbm.at[sd_v.at[b, 1]], sdd_v.at[b], dsem[b])

    def wait_gather(b):
        pltpu.make_async_copy(
            zp_hbm.at[sd_v.at[b, 0]], rows_v.at[b], gsem[b]).wait()
        pltpu.make_async_copy(
            sdd_hbm.at[sd_v.at[b, 1]], sdd_v.at[b], dsem[b]).wait()
        pltpu.make_async_copy(
            srelT_hbm.at[:, pl.ds(0, CH)], srel_v.at[b], rsem[b]).wait()

    def fire_scatter(b):
        pltpu.async_copy(rows_v.at[b], u_sh.at[sd_v.at[b, 1]], ssem[b],
                         add=True)

    def wait_scatter(b):
        pltpu.make_async_copy(
            rows_v.at[b], u_sh.at[sd_v.at[b, 1]], ssem[b]).wait()

    def compute(b):
        for g in range(CH // L):
            eidx = g * L + lane
            wh = []
            for h in range(HEADS):
                ss = plsc.load_gather(
                    rows_v.at[b], [eidx, jnp.full((L,), DIM + h, _i32)])
                sd = plsc.load_gather(
                    sdd_v.at[b], [eidx, jnp.full((L,), h, _i32)])
                x = ss + sd + srel_v[b, h, pl.ds(g * L, L)]
                x = jnp.where(x >= 0.0, x, 0.2 * x)
                w = jnp.exp(x)
                wh.append(w)
                plsc.store_scatter(
                    rows_v.at[b], [eidx, jnp.full((L,), DIM + h, _i32)], w)
            for e16 in range(L):
                e = g * L + e16
                for h in range(HEADS):
                    bc = _bcast16(wh[h], e16)
                    for q in range(DH // L):
                        col = h * DH + q * L
                        rows_v[b, e, pl.ds(col, L)] = (
                            rows_v[b, e, pl.ds(col, L)] * bc)

    plsc.subcore_barrier()
    stage(0, 0)
    fire_gather(0)

    def pair(t, carry):
        for b in (0, 1):
            j = 2 * t + b
            wait_gather(b)

            @pl.when(j + 1 < nchunk)
            def _prefetch():
                @pl.when(j >= 1)
                def _drain():
                    wait_scatter(1 - b)
                stage(j + 1, 1 - b)
                fire_gather(1 - b)

            compute(b)
            fire_scatter(b)
        return carry

    lax.fori_loop(0, nchunk // 2, pair, 0)
    wait_scatter(0)
    wait_scatter(1)
    plsc.subcore_barrier()
    pltpu.sync_copy(u_sh.at[pl.ds(s * ROWS_SUB, ROWS_SUB)],
                    u_out.at[c, pl.ds(s * ROWS_SUB, ROWS_SUB)])


def _edge_phase(zp, sd, srelT, sdd):
    mesh = plsc.VectorSubcoreMesh(
        core_axis_name="c", subcore_axis_name="s",
        num_cores=NC, num_subcores=NS)
    return pl.kernel(
        _edge_body,
        out_type=jax.ShapeDtypeStruct((NC, NPADU, RW), _f32),
        mesh=mesh,
        compiler_params=pltpu.CompilerParams(
            needs_layout_passes=False, use_tc_tiling_on_sc=False),
        scratch_types=[
            pltpu.VMEM((2, 2, CH), _i32),
            pltpu.VMEM((2, HEADS, CH), _f32),
            pltpu.VMEM((2, CH, SDW), _f32),
            pltpu.VMEM((2, CH, RW), _f32),
            pltpu.VMEM_SHARED((NPADU, RW), _f32),
            pltpu.SemaphoreType.DMA,
            pltpu.SemaphoreType.DMA,
            pltpu.SemaphoreType.DMA,
            pltpu.SemaphoreType.DMA,
            pltpu.SemaphoreType.DMA,
            pltpu.SemaphoreType.DMA,
            pltpu.SemaphoreType.DMA,
            pltpu.SemaphoreType.DMA,
        ],
    )(zp, sd, srelT, sdd)


def _final_body(u_ref, z_ref, pw_ref, pb_ref, ind_ref, out_ref):
    u = u_ref[0] + u_ref[1]
    d = u[:, DIM:DIM + HEADS]
    dinv = 1.0 / (d + 1e-6)
    dexp = jnp.dot(dinv, ind_ref[...], preferred_element_type=_f32)
    agg = u[:, :DIM] * dexp
    out_ref[...] = (
        jnp.dot(agg, pw_ref[...], preferred_element_type=_f32)
        + pb_ref[...] + z_ref[...])


def _finalize(u2, z, pw, pbr, ind):
    return pl.pallas_call(
        _final_body,
        grid=(N // BLKC,),
        in_specs=[
            pl.BlockSpec((NC, BLKC, RW), lambda i: (0, i, 0)),
            pl.BlockSpec((BLKC, DIM), lambda i: (i, 0)),
            pl.BlockSpec((DIM, DIM), lambda i: (0, 0)),
            pl.BlockSpec((1, DIM), lambda i: (0, 0)),
            pl.BlockSpec((HEADS, DIM), lambda i: (0, 0)),
        ],
        out_specs=pl.BlockSpec((BLKC, DIM), lambda i: (i, 0)),
        out_shape=jax.ShapeDtypeStruct((N, DIM), _f32),
    )(u2, z, pw, pbr, ind)


def kernel(z, edges, rel_offsets, W, a, rel_W1, rel_b1, rel_W2, rel_b2,
           proj_W, proj_b):
    wall = jnp.transpose(W, (1, 0, 2)).reshape(DIM, DIM)
    a_src, a_dst, a_rel = a[:, :DH], a[:, DH:2 * DH], a[:, 2 * DH:]
    va = jnp.concatenate([jnp.einsum('hdk,hk->dh', W, a_src),
                          jnp.einsum('hdk,hk->dh', W, a_dst)], axis=1)
    b2 = rel_W2 @ a_rel.T
    c2 = rel_b2 @ a_rel.T
    ind = jnp.repeat(jnp.eye(HEADS, dtype=_f32), DH, axis=1)

    roT = jnp.pad(rel_offsets, ((0, E_PAD - E), (0, 0))).T
    pad_i = jnp.zeros((E_PAD - E,), _i32)
    src = jnp.concatenate([edges[:, 0], pad_i])
    dst = jnp.concatenate([edges[:, 1], pad_i])
    sd = jnp.stack([src.reshape(-1, CH), dst.reshape(-1, CH)], axis=1)

    zp, sdd = _prep_nodes(z, wall, va)
    srelT = _prep_rel(roT, rel_W1.T, rel_b1.reshape(RP, 1),
                      b2.T, c2.reshape(HEADS, 1))
    u2 = _edge_phase(zp, sd, srelT, sdd)
    return _finalize(u2, z, proj_W, proj_b.reshape(1, DIM), ind)
